# baseline (speedup 1.0000x reference)
import jax
import jax.numpy as jnp
from jax import lax
from jax.experimental import pallas as pl
from jax.experimental.pallas import tpu as pltpu
from jax.experimental.pallas import tpu_sc as plsc

_N_TABLE = 200000
_EMBED_DIM = 64
_ROW_PAD = 128
_B_USERS = 4096
_B_ITEMS = 4096
_B_ALL = _B_USERS + _B_ITEMS

_NC = 2
_NS = 16
_NW = _NC * _NS

_CW = 512
_CW_SHIFT = _CW.bit_length() - 1
_NCHUNK = (_N_TABLE + _CW - 1) // _CW
_TAIL_CHUNK = _NCHUNK - 1
_TAIL_W = _N_TABLE - _TAIL_CHUNK * _CW
_TAIL_ALIGNED = (_TAIL_W // 128) * 128
_CPW = (_NCHUNK + _NW - 1) // _NW
_TRASH = _B_ALL
_TRASH2 = _B_ALL + 512


def _sc_gather_body(uids_hbm, iids_hbm, tableT_hbm, tail_hbm, emb_out,
                    ids_v, hit_ids, ch_lc, slab_v,
                    tail_v, stage_v, pos0_v, pos1_v,
                    sem0, sem1, semr):
    wid = lax.axis_index("s") * _NC + lax.axis_index("c")
    iota = lax.iota(jnp.int32, 16)
    zeros16 = jnp.zeros((16,), jnp.float32)

    for t in range(2):
        for s in range(16):
            for q in range(8):
                stage_v[t, s, pl.ds(q * 16, 16)] = zeros16

    pltpu.sync_copy(uids_hbm, ids_v.at[pl.ds(0, _B_USERS)])
    pltpu.sync_copy(iids_hbm, ids_v.at[pl.ds(_B_USERS, _B_ITEMS)])

    def gbody(k, cnt):
        v = ids_v[pl.ds(k * 16, 16)]
        m = ((v >> _CW_SHIFT) & (_NW - 1)) == wid
        mi = jnp.where(m, jnp.int32(1), jnp.int32(0))
        inc = lax.cumsum(mi, 0)
        dest = jnp.where(m, cnt + inc - 1, _TRASH)
        plsc.store_scatter(hit_ids, [dest], (v << 13) | (iota + k * 16))
        return cnt + lax.reduce_sum(mi, (0,))

    def start_chunk(k2):
        kg = k2 * _NW + wid
        parity = k2 % 2

        @pl.when(kg < _NCHUNK)
        def _():
            cbase = pl.multiple_of(kg * _CW, _CW)

            @pl.when((kg != _TAIL_CHUNK) & (parity == 0))
            def _():
                pltpu.make_async_copy(
                    tableT_hbm.at[:, pl.ds(cbase, _CW)],
                    slab_v.at[0], sem0).start()

            @pl.when((kg != _TAIL_CHUNK) & (parity == 1))
            def _():
                pltpu.make_async_copy(
                    tableT_hbm.at[:, pl.ds(cbase, _CW)],
                    slab_v.at[1], sem1).start()

            @pl.when(kg == _TAIL_CHUNK)
            def _():
                tsrc = tableT_hbm.at[:, pl.ds(
                    pl.multiple_of(_TAIL_CHUNK * _CW, _CW), _TAIL_ALIGNED)]

                @pl.when(parity == 0)
                def _():
                    pltpu.make_async_copy(
                        tsrc, slab_v.at[0, :, pl.ds(0, _TAIL_ALIGNED)],
                        sem0).start()

                @pl.when(parity == 1)
                def _():
                    pltpu.make_async_copy(
                        tsrc, slab_v.at[1, :, pl.ds(0, _TAIL_ALIGNED)],
                        sem1).start()

                pltpu.sync_copy(tail_hbm, tail_v)

    def wait_chunk(k2):
        kg = k2 * _NW + wid
        parity = k2 % 2

        @pl.when(kg < _NCHUNK)
        def _():
            cbase = pl.multiple_of(kg * _CW, _CW)

            @pl.when((kg != _TAIL_CHUNK) & (parity == 0))
            def _():
                pltpu.make_async_copy(
                    tableT_hbm.at[:, pl.ds(cbase, _CW)],
                    slab_v.at[0], sem0).wait()

            @pl.when((kg != _TAIL_CHUNK) & (parity == 1))
            def _():
                pltpu.make_async_copy(
                    tableT_hbm.at[:, pl.ds(cbase, _CW)],
                    slab_v.at[1], sem1).wait()

            @pl.when(kg == _TAIL_CHUNK)
            def _():
                tsrc = tableT_hbm.at[:, pl.ds(
                    pl.multiple_of(_TAIL_CHUNK * _CW, _CW), _TAIL_ALIGNED)]

                @pl.when(parity == 0)
                def _():
                    pltpu.make_async_copy(
                        tsrc, slab_v.at[0, :, pl.ds(0, _TAIL_ALIGNED)],
                        sem0).wait()

                @pl.when(parity == 1)
                def _():
                    pltpu.make_async_copy(
                        tsrc, slab_v.at[1, :, pl.ds(0, _TAIL_ALIGNED)],
                        sem1).wait()

    start_chunk(jnp.int32(0))
    start_chunk(jnp.int32(1))

    cnt = lax.fori_loop(0, _B_ALL // 16, gbody, jnp.int32(0))

    def chunk_body(k2, _):
        kg = k2 * _NW + wid
        wait_chunk(k2)

        @pl.when(kg < _NCHUNK)
        def _():
            cbase = pl.multiple_of(kg * _CW, _CW)
            parity = k2 % 2
            pv16 = jnp.full((16,), parity, jnp.int32)

            def rbody(m, scnt):
                pk = hit_ids[pl.ds(m * 16, 16)]
                v = pk >> 13
                valid = iota < (cnt - m * 16)
                mm = (v >= cbase) & (v < cbase + _CW) & valid
                mi = jnp.where(mm, jnp.int32(1), jnp.int32(0))
                inc = lax.cumsum(mi, 0)
                dest = jnp.where(mm, scnt + inc - 1, _TRASH2)
                plsc.store_scatter(ch_lc, [dest], pk - (cbase << 13))
                return scnt + lax.reduce_sum(mi, (0,))

            scnt = lax.fori_loop(0, (cnt + 15) // 16, rbody, jnp.int32(0))

            @pl.when(kg != _TAIL_CHUNK)
            def _():
                nblk = (scnt + 15) // 16

                def blk(b, _):
                    par = b % 2

                    @pl.when(b >= 2)
                    def _():
                        @pl.when(par == 0)
                        def _():
                            pltpu.make_async_copy(
                                emb_out.at[pl.ds(0, 16), :],
                                stage_v.at[0], semr).wait()

                        @pl.when(par == 1)
                        def _():
                            pltpu.make_async_copy(
                                emb_out.at[pl.ds(0, 16), :],
                                stage_v.at[1], semr).wait()

                    pkv = ch_lc[pl.ds(b * 16, 16)]
                    lcv = (pkv >> 13) & (_CW - 1)
                    validm = iota < (scnt - b * 16)
                    posv = jnp.where(validm, pkv & 8191, jnp.int32(_B_ALL))
                    parv = jnp.full((16,), par, jnp.int32)
                    for d in range(_EMBED_DIM):
                        dv = jnp.full((16,), d, jnp.int32)
                        vals = plsc.load_gather(slab_v, [pv16, dv, lcv])
                        plsc.store_scatter(stage_v, [parv, iota, dv], vals)

                    @pl.when(par == 0)
                    def _():
                        pos0_v[...] = posv
                        pltpu.make_async_copy(
                            stage_v.at[0], emb_out.at[pos0_v], semr).start()

                    @pl.when(par == 1)
                    def _():
                        pos1_v[...] = posv
                        pltpu.make_async_copy(
                            stage_v.at[1], emb_out.at[pos1_v], semr).start()

                    return 0

                lax.fori_loop(0, nblk, blk, 0)

                def fdrain(d, _):
                    @pl.when(d % 2 == 0)
                    def _():
                        pltpu.make_async_copy(
                            emb_out.at[pl.ds(0, 16), :],
                            stage_v.at[0], semr).wait()

                    @pl.when(d % 2 == 1)
                    def _():
                        pltpu.make_async_copy(
                            emb_out.at[pl.ds(0, 16), :],
                            stage_v.at[1], semr).wait()

                    return 0

                lax.fori_loop(0, jnp.minimum(nblk, jnp.int32(2)), fdrain, 0)

            @pl.when(kg == _TAIL_CHUNK)
            def _():
                nblk = (scnt + 15) // 16

                def blk(b, _):
                    par = b % 2

                    @pl.when(b >= 2)
                    def _():
                        @pl.when(par == 0)
                        def _():
                            pltpu.make_async_copy(
                                emb_out.at[pl.ds(0, 16), :],
                                stage_v.at[0], semr).wait()

                        @pl.when(par == 1)
                        def _():
                            pltpu.make_async_copy(
                                emb_out.at[pl.ds(0, 16), :],
                                stage_v.at[1], semr).wait()

                    pkv = ch_lc[pl.ds(b * 16, 16)]
                    lcr = pkv >> 13
                    lcv = lcr & (_CW - 1)
                    ltv = jnp.clip(lcr - _TAIL_ALIGNED, 0,
                                   _TAIL_W - _TAIL_ALIGNED - 1)
                    sel = lcr < _TAIL_ALIGNED
                    validm = iota < (scnt - b * 16)
                    posv = jnp.where(validm, pkv & 8191, jnp.int32(_B_ALL))
                    parv = jnp.full((16,), par, jnp.int32)
                    for d in range(_EMBED_DIM):
                        dv = jnp.full((16,), d, jnp.int32)
                        vq = plsc.load_gather(slab_v, [pv16, dv, lcv])
                        vt = plsc.load_gather(tail_v, [dv, ltv])
                        plsc.store_scatter(stage_v, [parv, iota, dv],
                                           jnp.where(sel, vq, vt))

                    @pl.when(par == 0)
                    def _():
                        pos0_v[...] = posv
                        pltpu.make_async_copy(
                            stage_v.at[0], emb_out.at[pos0_v], semr).start()

                    @pl.when(par == 1)
                    def _():
                        pos1_v[...] = posv
                        pltpu.make_async_copy(
                            stage_v.at[1], emb_out.at[pos1_v], semr).start()

                    return 0

                lax.fori_loop(0, nblk, blk, 0)

                def fdrain(d, _):
                    @pl.when(d % 2 == 0)
                    def _():
                        pltpu.make_async_copy(
                            emb_out.at[pl.ds(0, 16), :],
                            stage_v.at[0], semr).wait()

                    @pl.when(d % 2 == 1)
                    def _():
                        pltpu.make_async_copy(
                            emb_out.at[pl.ds(0, 16), :],
                            stage_v.at[1], semr).wait()

                    return 0

                lax.fori_loop(0, jnp.minimum(nblk, jnp.int32(2)), fdrain, 0)

        start_chunk(k2 + 2)
        return 0

    lax.fori_loop(0, _CPW, chunk_body, 0)


_sc_gather = pl.kernel(
    _sc_gather_body,
    out_type=jax.ShapeDtypeStruct((_B_ALL + 8, _ROW_PAD), jnp.float32),
    mesh=plsc.VectorSubcoreMesh(core_axis_name="c", subcore_axis_name="s"),
    scratch_types=(
        pltpu.VMEM((_B_ALL,), jnp.int32),
        pltpu.VMEM((_B_ALL + 16,), jnp.int32),
        pltpu.VMEM((_B_ALL + 528,), jnp.int32),
        pltpu.VMEM((2, _EMBED_DIM, _CW), jnp.float32),
        pltpu.VMEM((_EMBED_DIM, _TAIL_W - _TAIL_ALIGNED), jnp.float32),
        pltpu.VMEM((2, 16, _ROW_PAD), jnp.float32),
        pltpu.VMEM((16,), jnp.int32),
        pltpu.VMEM((16,), jnp.int32),
        pltpu.SemaphoreType.DMA,
        pltpu.SemaphoreType.DMA,
        pltpu.SemaphoreType.DMA,
    ),
    compiler_params=pltpu.CompilerParams(needs_layout_passes=False),
)

_BM = 512
_BN = 4096


def _mm_body(rid_ref, u_ref, i_ref, r_ref, o_ref):
    r = r_ref[pl.ds(rid_ref[0], 1), :]
    u = u_ref[...] + r
    o_ref[...] = lax.dot_general(
        u, i_ref[...], (((1,), (1,)), ((), ())),
        preferred_element_type=jnp.float32)


@jax.jit
def _tc_matmul(rid, comb, relation_embed_pad):
    grid = (_B_USERS // _BM, _B_ITEMS // _BN)
    return pl.pallas_call(
        _mm_body,
        grid_spec=pltpu.PrefetchScalarGridSpec(
            num_scalar_prefetch=1,
            grid=grid,
            in_specs=[
                pl.BlockSpec((_BM, _ROW_PAD), lambda i, j, rid: (i, 0)),
                pl.BlockSpec((_BN, _ROW_PAD),
                             lambda i, j, rid: (j + _B_USERS // _BN, 0)),
                pl.BlockSpec((16, _ROW_PAD), lambda i, j, rid: (0, 0)),
            ],
            out_specs=pl.BlockSpec((_BM, _BN), lambda i, j, rid: (i, j)),
        ),
        out_shape=jax.ShapeDtypeStruct((_B_USERS, _B_ITEMS), jnp.float32),
        compiler_params=pltpu.CompilerParams(
            dimension_semantics=("parallel", "parallel")),
    )(rid, comb, comb, relation_embed_pad)


def kernel(user_ids, item_ids, relation_id, is_train, relation_embed,
           entity_user_embed):
    del is_train
    uids = user_ids.astype(jnp.int32)
    iids = item_ids.astype(jnp.int32)
    rid = relation_id.astype(jnp.int32)
    table_t = entity_user_embed.T
    tail = lax.slice(table_t, (0, _TAIL_CHUNK * _CW + _TAIL_ALIGNED),
                     (_EMBED_DIM, _N_TABLE))
    comb = _sc_gather(uids, iids, table_t, tail)
    r_pad = jnp.pad(relation_embed, ((0, 0), (0, _ROW_PAD - _EMBED_DIM)))
    return _tc_matmul(rid, comb, r_pad)


# speedup vs baseline: 2.3298x; 2.3298x over previous
import jax
import jax.numpy as jnp
from jax import lax
from jax.experimental import pallas as pl
from jax.experimental.pallas import tpu as pltpu
from jax.experimental.pallas import tpu_sc as plsc

_N_TABLE = 200000
_EMBED_DIM = 64
_ROW_PAD = 128
_B_USERS = 4096
_B_ITEMS = 4096
_B_ALL = _B_USERS + _B_ITEMS

_NC = 2
_NS = 16
_NW = _NC * _NS

_CW = 512
_CW_SHIFT = _CW.bit_length() - 1
_NCHUNK = (_N_TABLE + _CW - 1) // _CW
_TAIL_CHUNK = _NCHUNK - 1
_TAIL_W = _N_TABLE - _TAIL_CHUNK * _CW
_TAIL_ALIGNED = (_TAIL_W // 128) * 128
_CPW = (_NCHUNK + _NW - 1) // _NW
_TRASH = _B_ALL
_TRASH2 = _B_ALL + 512


def _sc_gather_body(uids_hbm, iids_hbm, tableT_hbm, tail_hbm, emb_out,
                    ids_v, hit_ids, ch_lc, slab_v,
                    tail_v, row_ring, sem0, sem1, semr):
    wid = lax.axis_index("s") * _NC + lax.axis_index("c")
    iota = lax.iota(jnp.int32, 16)
    zeros16 = jnp.zeros((16,), jnp.float32)

    for s in range(8):
        for q in range(8):
            row_ring[s, pl.ds(q * 16, 16)] = zeros16

    pltpu.sync_copy(uids_hbm, ids_v.at[pl.ds(0, _B_USERS)])
    pltpu.sync_copy(iids_hbm, ids_v.at[pl.ds(_B_USERS, _B_ITEMS)])

    def gbody(k, cnt):
        v = ids_v[pl.ds(k * 16, 16)]
        m = ((v >> _CW_SHIFT) & (_NW - 1)) == wid
        mi = jnp.where(m, jnp.int32(1), jnp.int32(0))
        inc = lax.cumsum(mi, 0)
        dest = jnp.where(m, cnt + inc - 1, _TRASH)
        plsc.store_scatter(hit_ids, [dest], (v << 13) | (iota + k * 16))
        return cnt + lax.reduce_sum(mi, (0,))

    def start_chunk(k2):
        kg = k2 * _NW + wid
        parity = k2 % 2

        @pl.when(kg < _NCHUNK)
        def _():
            cbase = pl.multiple_of(kg * _CW, _CW)

            @pl.when((kg != _TAIL_CHUNK) & (parity == 0))
            def _():
                pltpu.make_async_copy(
                    tableT_hbm.at[:, pl.ds(cbase, _CW)],
                    slab_v.at[0], sem0).start()

            @pl.when((kg != _TAIL_CHUNK) & (parity == 1))
            def _():
                pltpu.make_async_copy(
                    tableT_hbm.at[:, pl.ds(cbase, _CW)],
                    slab_v.at[1], sem1).start()

            @pl.when(kg == _TAIL_CHUNK)
            def _():
                tsrc = tableT_hbm.at[:, pl.ds(
                    pl.multiple_of(_TAIL_CHUNK * _CW, _CW), _TAIL_ALIGNED)]

                @pl.when(parity == 0)
                def _():
                    pltpu.make_async_copy(
                        tsrc, slab_v.at[0, :, pl.ds(0, _TAIL_ALIGNED)],
                        sem0).start()

                @pl.when(parity == 1)
                def _():
                    pltpu.make_async_copy(
                        tsrc, slab_v.at[1, :, pl.ds(0, _TAIL_ALIGNED)],
                        sem1).start()

                pltpu.sync_copy(tail_hbm, tail_v)

    def wait_chunk(k2):
        kg = k2 * _NW + wid
        parity = k2 % 2

        @pl.when(kg < _NCHUNK)
        def _():
            cbase = pl.multiple_of(kg * _CW, _CW)

            @pl.when((kg != _TAIL_CHUNK) & (parity == 0))
            def _():
                pltpu.make_async_copy(
                    tableT_hbm.at[:, pl.ds(cbase, _CW)],
                    slab_v.at[0], sem0).wait()

            @pl.when((kg != _TAIL_CHUNK) & (parity == 1))
            def _():
                pltpu.make_async_copy(
                    tableT_hbm.at[:, pl.ds(cbase, _CW)],
                    slab_v.at[1], sem1).wait()

            @pl.when(kg == _TAIL_CHUNK)
            def _():
                tsrc = tableT_hbm.at[:, pl.ds(
                    pl.multiple_of(_TAIL_CHUNK * _CW, _CW), _TAIL_ALIGNED)]

                @pl.when(parity == 0)
                def _():
                    pltpu.make_async_copy(
                        tsrc, slab_v.at[0, :, pl.ds(0, _TAIL_ALIGNED)],
                        sem0).wait()

                @pl.when(parity == 1)
                def _():
                    pltpu.make_async_copy(
                        tsrc, slab_v.at[1, :, pl.ds(0, _TAIL_ALIGNED)],
                        sem1).wait()

    start_chunk(jnp.int32(0))
    start_chunk(jnp.int32(1))

    cnt = lax.fori_loop(0, _B_ALL // 16, gbody, jnp.int32(0))

    def chunk_body(k2, _):
        kg = k2 * _NW + wid
        wait_chunk(k2)

        @pl.when(kg < _NCHUNK)
        def _():
            cbase = pl.multiple_of(kg * _CW, _CW)
            parity = k2 % 2
            pv16 = jnp.full((16,), parity, jnp.int32)

            def rbody(m, scnt):
                pk = hit_ids[pl.ds(m * 16, 16)]
                v = pk >> 13
                valid = iota < (cnt - m * 16)
                mm = (v >= cbase) & (v < cbase + _CW) & valid
                mi = jnp.where(mm, jnp.int32(1), jnp.int32(0))
                inc = lax.cumsum(mi, 0)
                dest = jnp.where(mm, scnt + inc - 1, _TRASH2)
                plsc.store_scatter(ch_lc, [dest], pk - (cbase << 13))
                return scnt + lax.reduce_sum(mi, (0,))

            scnt = lax.fori_loop(0, (cnt + 15) // 16, rbody, jnp.int32(0))

            def finish(h, p, _):
                pltpu.make_async_copy(
                    row_ring.at[h % 8],
                    emb_out.at[pl.ds(p * _ROW_PAD, _ROW_PAD)],
                    semr).start()
                return 0

            def dbody(d, _):
                pltpu.make_async_copy(
                    emb_out.at[pl.ds(0, _ROW_PAD)],
                    row_ring.at[d % 8], semr).wait()
                return 0

            @pl.when(kg != _TAIL_CHUNK)
            def _():
                def hbody(h, _):
                    slot = h % 8

                    @pl.when(h >= 8)
                    def _():
                        pltpu.make_async_copy(
                            emb_out.at[pl.ds(0, _ROW_PAD)],
                            row_ring.at[slot], semr).wait()

                    pk = ch_lc[pl.ds(h, 16)][0]
                    lcv = jnp.full((16,), pk >> 13, jnp.int32)
                    for q in range(4):
                        row_ring[slot, pl.ds(q * 16, 16)] = plsc.load_gather(
                            slab_v, [pv16, iota + q * 16, lcv])
                    return finish(h, pk & 8191, 0)

                lax.fori_loop(0, scnt, hbody, 0)
                lax.fori_loop(0, jnp.minimum(scnt, jnp.int32(8)), dbody, 0)

            @pl.when(kg == _TAIL_CHUNK)
            def _():
                def hbody(h, _):
                    slot = h % 8

                    @pl.when(h >= 8)
                    def _():
                        pltpu.make_async_copy(
                            emb_out.at[pl.ds(0, _ROW_PAD)],
                            row_ring.at[slot], semr).wait()

                    pk = ch_lc[pl.ds(h, 16)][0]
                    lc = pk >> 13
                    use_slab = lc < _TAIL_ALIGNED
                    lcv = jnp.full((16,), jnp.where(use_slab, lc, 0),
                                   jnp.int32)
                    ltv = jnp.full(
                        (16,),
                        jnp.clip(lc - _TAIL_ALIGNED, 0, _TAIL_W
                                 - _TAIL_ALIGNED - 1), jnp.int32)
                    sel = jnp.full((16,), use_slab, jnp.bool_)
                    for q in range(4):
                        colq = plsc.load_gather(
                            slab_v, [pv16, iota + q * 16, lcv])
                        colt = plsc.load_gather(
                            tail_v, [iota + q * 16, ltv])
                        row_ring[slot, pl.ds(q * 16, 16)] = jnp.where(
                            sel, colq, colt)
                    return finish(h, pk & 8191, 0)

                lax.fori_loop(0, scnt, hbody, 0)
                lax.fori_loop(0, jnp.minimum(scnt, jnp.int32(8)), dbody, 0)

        start_chunk(k2 + 2)
        return 0

    lax.fori_loop(0, _CPW, chunk_body, 0)


_sc_gather = pl.kernel(
    _sc_gather_body,
    out_type=jax.ShapeDtypeStruct((_B_ALL * _ROW_PAD,), jnp.float32),
    mesh=plsc.VectorSubcoreMesh(core_axis_name="c", subcore_axis_name="s"),
    scratch_types=(
        pltpu.VMEM((_B_ALL,), jnp.int32),
        pltpu.VMEM((_B_ALL + 16,), jnp.int32),
        pltpu.VMEM((_B_ALL + 528,), jnp.int32),
        pltpu.VMEM((2, _EMBED_DIM, _CW), jnp.float32),
        pltpu.VMEM((_EMBED_DIM, _TAIL_W - _TAIL_ALIGNED), jnp.float32),
        pltpu.VMEM((8, _ROW_PAD), jnp.float32),
        pltpu.SemaphoreType.DMA,
        pltpu.SemaphoreType.DMA,
        pltpu.SemaphoreType.DMA,
    ),
    compiler_params=pltpu.CompilerParams(needs_layout_passes=False),
)

_BM = 512
_BN = 4096


def _mm_body(rid_ref, u_ref, i_ref, r_ref, o_ref):
    r = r_ref[pl.ds(rid_ref[0], 1), :]
    u = u_ref[...] + r
    o_ref[...] = lax.dot_general(
        u, i_ref[...], (((1,), (1,)), ((), ())),
        preferred_element_type=jnp.float32)


@jax.jit
def _tc_matmul(rid, comb, relation_embed_pad):
    grid = (_B_USERS // _BM, _B_ITEMS // _BN)
    return pl.pallas_call(
        _mm_body,
        grid_spec=pltpu.PrefetchScalarGridSpec(
            num_scalar_prefetch=1,
            grid=grid,
            in_specs=[
                pl.BlockSpec((_BM, _ROW_PAD), lambda i, j, rid: (i, 0)),
                pl.BlockSpec((_BN, _ROW_PAD),
                             lambda i, j, rid: (j + _B_USERS // _BN, 0)),
                pl.BlockSpec((16, _ROW_PAD), lambda i, j, rid: (0, 0)),
            ],
            out_specs=pl.BlockSpec((_BM, _BN), lambda i, j, rid: (i, j)),
        ),
        out_shape=jax.ShapeDtypeStruct((_B_USERS, _B_ITEMS), jnp.float32),
        compiler_params=pltpu.CompilerParams(
            dimension_semantics=("parallel", "parallel")),
    )(rid, comb, comb, relation_embed_pad)


def kernel(user_ids, item_ids, relation_id, is_train, relation_embed,
           entity_user_embed):
    del is_train
    uids = user_ids.astype(jnp.int32)
    iids = item_ids.astype(jnp.int32)
    rid = relation_id.astype(jnp.int32)
    table_t = entity_user_embed.T
    tail = lax.slice(table_t, (0, _TAIL_CHUNK * _CW + _TAIL_ALIGNED),
                     (_EMBED_DIM, _N_TABLE))
    emb1d = _sc_gather(uids, iids, table_t, tail)
    comb = emb1d.reshape(_B_ALL, _ROW_PAD)
    r_pad = jnp.pad(relation_embed, ((0, 0), (0, _ROW_PAD - _EMBED_DIM)))
    return _tc_matmul(rid, comb, r_pad)
